# Initial kernel scaffold; baseline (speedup 1.0000x reference)
import jax
import jax.numpy as jnp
from jax.experimental import pallas as pl


def kernel(x, edge_index, edge_attr, node_W, node_b, edge_W, edge_b, Wq, bq, Wk, bk, Wv, bv, We, Wskip, bskip, ln_g, ln_b, out_W, out_b):
    raise NotImplementedError("write your pallas kernel here")


# speedup vs baseline: 28.4038x; 28.4038x over previous
import functools

import jax
import jax.numpy as jnp
from jax import lax
from jax.experimental import pallas as pl
from jax.experimental.pallas import tpu as pltpu
from jax.experimental.pallas import tpu_sc as plsc

N = 10000
E = 320000
D = 128
DE = 16
H = 8
C = 16
SCALE = 0.25

NC = 2
NS = 16
NW = NC * NS
EPW = E // NW
B = 80
NCH = EPW // B
N_PAD = 10240
RPT = N_PAD // NS


def _proj_body(x_ref, w_ref, b_ref, o_ref):
    o_ref[...] = jnp.dot(x_ref[...], w_ref[...],
                         preferred_element_type=jnp.float32) + b_ref[...]


def _proj(x, w, b2d):
    bn = 2000
    grid = N // bn
    row = pl.BlockSpec((bn, D), lambda i: (i, 0))
    return pl.pallas_call(
        _proj_body,
        grid=(grid,),
        in_specs=[row, pl.BlockSpec((D, D), lambda i: (0, 0)),
                  pl.BlockSpec((1, D), lambda i: (0, 0))],
        out_specs=row,
        out_shape=jax.ShapeDtypeStruct((N, D), jnp.float32),
    )(x, w, b2d)


def _qkvs_body(h_ref, wq_ref, wk_ref, wv_ref, ws_ref, bq_ref, bk_ref, bv_ref,
               bs_ref, q_ref, k_ref, v_ref, s_ref):
    hb = h_ref[...]
    q_ref[...] = jnp.dot(hb, wq_ref[...], preferred_element_type=jnp.float32) + bq_ref[...]
    k_ref[...] = jnp.dot(hb, wk_ref[...], preferred_element_type=jnp.float32) + bk_ref[...]
    v_ref[...] = jnp.dot(hb, wv_ref[...], preferred_element_type=jnp.float32) + bv_ref[...]
    s_ref[...] = jnp.dot(hb, ws_ref[...], preferred_element_type=jnp.float32) + bs_ref[...]


def _qkvs(h, wq, wk, wv, ws, bq, bk, bv, bs):
    bn = 2000
    grid = N // bn
    row = pl.BlockSpec((bn, D), lambda i: (i, 0))
    mat = pl.BlockSpec((D, D), lambda i: (0, 0))
    bias = pl.BlockSpec((1, D), lambda i: (0, 0))
    out = jax.ShapeDtypeStruct((N, D), jnp.float32)
    return pl.pallas_call(
        _qkvs_body,
        grid=(grid,),
        in_specs=[row, mat, mat, mat, mat, bias, bias, bias, bias],
        out_specs=[row, row, row, row],
        out_shape=[out, out, out, out],
    )(h, wq, wk, wv, ws, bq, bk, bv, bs)


def _ee_body(ea_ref, ew_ref, we_ref, eb_ref, ee_ref):
    w2 = jnp.dot(ew_ref[...], we_ref[...], preferred_element_type=jnp.float32)
    b2 = jnp.dot(eb_ref[...], we_ref[...], preferred_element_type=jnp.float32)
    ee_ref[...] = jnp.dot(ea_ref[...], w2, preferred_element_type=jnp.float32) + b2


def _ee(edge_attr, edge_W, edge_b2d, We_l):
    be = 4000
    grid = E // be
    return pl.pallas_call(
        _ee_body,
        grid=(grid,),
        in_specs=[
            pl.BlockSpec((be, DE), lambda i: (i, 0)),
            pl.BlockSpec((DE, D), lambda i: (0, 0)),
            pl.BlockSpec((D, D), lambda i: (0, 0)),
            pl.BlockSpec((1, D), lambda i: (0, 0)),
        ],
        out_specs=pl.BlockSpec((be, D), lambda i: (i, 0)),
        out_shape=jax.ShapeDtypeStruct((E, D), jnp.float32),
    )(edge_attr, edge_W, We_l, edge_b2d)


def _edge_body(qi_ref, kj_ref, vj_ref, ee_ref, g_ref, s_ref, m_ref, w_ref):
    eeb = ee_ref[...]
    kj = kj_ref[...] + eeb
    vj = vj_ref[...] + eeb
    t = qi_ref[...] * kj
    alpha = jnp.dot(t, g_ref[...], preferred_element_type=jnp.float32)
    ex = jnp.exp(alpha * SCALE)
    exb = jnp.dot(ex, s_ref[...], preferred_element_type=jnp.float32)
    w_ref[...] = exb
    m_ref[...] = exb * vj


def _edge_compute(qi, kj, vj, ee, gsel, ssel):
    be = 2000
    grid = E // be
    row = pl.BlockSpec((be, D), lambda i: (i, 0))
    return pl.pallas_call(
        _edge_body,
        grid=(grid,),
        in_specs=[row, row, row, row,
                  pl.BlockSpec((D, H), lambda i: (0, 0)),
                  pl.BlockSpec((H, D), lambda i: (0, 0))],
        out_specs=[row, row],
        out_shape=[jax.ShapeDtypeStruct((E, D), jnp.float32),
                   jax.ShapeDtypeStruct((E, D), jnp.float32)],
    )(qi, kj, vj, ee, gsel, ssel)


def _norm_body(a0_ref, a1_ref, d0_ref, d1_ref, skip_ref, h_ref, o_ref):
    denb = d0_ref[...] + d1_ref[...]
    agg = (a0_ref[...] + a1_ref[...]) / (denb + 1e-16) + skip_ref[...]
    o_ref[...] = h_ref[...] + jnp.maximum(agg, 0.0)


def _norm_update(acc0, acc1, den0, den1, skip, h):
    bn = 2000
    grid = N // bn
    row = pl.BlockSpec((bn, D), lambda i: (i, 0))
    return pl.pallas_call(
        _norm_body,
        grid=(grid,),
        in_specs=[row, row, row, row, row, row],
        out_specs=row,
        out_shape=jax.ShapeDtypeStruct((N, D), jnp.float32),
    )(acc0, acc1, den0, den1, skip, h)


def _final_body(h_ref, g_ref, b_ref, ow_ref, ob_ref, o_ref, acc_ref):
    i = pl.program_id(0)
    nblk = pl.num_programs(0)
    hb = h_ref[...]
    mu = jnp.mean(hb, axis=1, keepdims=True)
    var = jnp.mean((hb - mu) ** 2, axis=1, keepdims=True)
    hn = (hb - mu) / jnp.sqrt(var + 1e-5) * g_ref[...] + b_ref[...]
    psum = jnp.sum(hn, axis=0, keepdims=True)

    @pl.when(i == 0)
    def _():
        acc_ref[...] = jnp.zeros_like(acc_ref)

    acc_ref[...] += psum

    @pl.when(i == nblk - 1)
    def _():
        o_ref[...] = jnp.dot(acc_ref[...] * (1.0 / N), ow_ref[...],
                             preferred_element_type=jnp.float32) + ob_ref[...]


def _final(h, ln_g2d, ln_b2d, out_W, out_b2d):
    bn = 2000
    grid = N // bn
    return pl.pallas_call(
        _final_body,
        grid=(grid,),
        in_specs=[pl.BlockSpec((bn, D), lambda i: (i, 0)),
                  pl.BlockSpec((1, D), lambda i: (0, 0)),
                  pl.BlockSpec((1, D), lambda i: (0, 0)),
                  pl.BlockSpec((D, D), lambda i: (0, 0)),
                  pl.BlockSpec((1, D), lambda i: (0, 0))],
        out_specs=pl.BlockSpec((1, D), lambda i: (0, 0)),
        out_shape=jax.ShapeDtypeStruct((1, D), jnp.float32),
        scratch_shapes=[pltpu.VMEM((1, D), jnp.float32)],
    )(h, ln_g2d, ln_b2d, out_W, out_b2d)


def _sc_gather_body(kT, vT, qT, src, dst, kj_out, vj_out, qi_out,
                    idx_s, idx_d, kb, vb, qb, sem):
    cid = lax.axis_index("c")
    sid = lax.axis_index("s")
    wid = cid * NS + sid
    base = wid * EPW

    def chunk(ch, carry):
        eb = base + ch * B
        pltpu.sync_copy(src.at[pl.ds(eb, B)], idx_s)
        pltpu.sync_copy(dst.at[pl.ds(eb, B)], idx_d)
        d1 = pltpu.async_copy(kT.at[idx_s], kb, sem)
        d2 = pltpu.async_copy(vT.at[idx_s], vb, sem)
        d3 = pltpu.async_copy(qT.at[idx_d], qb, sem)
        d1.wait()
        d2.wait()
        d3.wait()
        pltpu.sync_copy(kb, kj_out.at[pl.ds(eb, B)])
        pltpu.sync_copy(vb, vj_out.at[pl.ds(eb, B)])
        pltpu.sync_copy(qb, qi_out.at[pl.ds(eb, B)])
        return carry

    lax.fori_loop(0, NCH, chunk, 0)


def _sc_gather(kT, vT, qT, src, dst):
    mesh = plsc.VectorSubcoreMesh(core_axis_name="c", subcore_axis_name="s")
    out = jax.ShapeDtypeStruct((E, D), jnp.float32)
    f = pl.kernel(
        _sc_gather_body,
        out_type=[out, out, out],
        mesh=mesh,
        scratch_types=[
            pltpu.VMEM((B,), jnp.int32),
            pltpu.VMEM((B,), jnp.int32),
            pltpu.VMEM((B, D), jnp.float32),
            pltpu.VMEM((B, D), jnp.float32),
            pltpu.VMEM((B, D), jnp.float32),
            pltpu.SemaphoreType.DMA,
        ],
    )
    return f(kT, vT, qT, src, dst)


def _sc_scatter_body(msg, w8, dst, znode, acc_out, den_out,
                     tab_sh, idx, mb):
    cid = lax.axis_index("c")
    sid = lax.axis_index("s")
    wid = cid * NS + sid
    base = wid * EPW
    r0 = sid * RPT
    off = cid * N_PAD + r0

    def accumulate(src_arr, out_arr):
        pltpu.sync_copy(znode.at[pl.ds(r0, RPT)], tab_sh.at[pl.ds(r0, RPT)])
        plsc.subcore_barrier()

        def chunk(ch, carry):
            eb = base + ch * B
            pltpu.sync_copy(dst.at[pl.ds(eb, B)], idx.at[0])
            pltpu.sync_copy(src_arr.at[pl.ds(eb, B)], mb)
            pltpu.sync_copy(mb, tab_sh.at[idx.at[0]], add=True)
            return carry

        lax.fori_loop(0, NCH, chunk, 0)
        plsc.subcore_barrier()
        pltpu.sync_copy(tab_sh.at[pl.ds(r0, RPT)], out_arr.at[pl.ds(off, RPT)])
        plsc.subcore_barrier()

    accumulate(msg, acc_out)
    accumulate(w8, den_out)


def _sc_scatter(msg, w8, dst, znode):
    mesh = plsc.VectorSubcoreMesh(core_axis_name="c", subcore_axis_name="s")
    f = pl.kernel(
        _sc_scatter_body,
        out_type=[jax.ShapeDtypeStruct((NC * N_PAD, D), jnp.float32),
                  jax.ShapeDtypeStruct((NC * N_PAD, D), jnp.float32)],
        mesh=mesh,
        scratch_types=[
            pltpu.VMEM_SHARED((N_PAD, D), jnp.float32),
            pltpu.VMEM((1, B), jnp.int32),
            pltpu.VMEM((B, D), jnp.float32),
        ],
    )
    accP, denP = f(msg, w8, dst, znode)
    return accP.reshape(NC, N_PAD, D), denP.reshape(NC, N_PAD, D)


def kernel(x, edge_index, edge_attr, node_W, node_b, edge_W, edge_b, Wq, bq,
           Wk, bk, Wv, bv, We, Wskip, bskip, ln_g, ln_b, out_W, out_b):
    src = edge_index[0]
    dst = edge_index[1]

    lane = jnp.arange(D, dtype=jnp.int32)
    head = jnp.arange(H, dtype=jnp.int32)
    gsel = (lane[:, None] // C == head[None, :]).astype(jnp.float32)
    ssel = gsel.T.copy()

    znode = jnp.zeros((N_PAD, D), jnp.float32)

    node_b2 = node_b.reshape(1, D)
    edge_b2 = edge_b.reshape(1, D)

    h = _proj(x, node_W, node_b2)

    for l in range(2):
        q, k, v, skip = _qkvs(h, Wq[l], Wk[l], Wv[l], Wskip[l],
                              bq[l].reshape(1, D), bk[l].reshape(1, D),
                              bv[l].reshape(1, D), bskip[l].reshape(1, D))
        ee = _ee(edge_attr, edge_W, edge_b2, We[l])
        kj, vj, qi = _sc_gather(k, v, q, src, dst)
        msg, w8 = _edge_compute(qi, kj, vj, ee, gsel, ssel)
        accP, denP = _sc_scatter(msg, w8, dst, znode)
        h = _norm_update(accP[0, :N], accP[1, :N], denP[0, :N], denP[1, :N],
                         skip, h)

    return _final(h, ln_g.reshape(1, D), ln_b.reshape(1, D), out_W,
                  out_b.reshape(1, D))


# speedup vs baseline: 36.6972x; 1.2920x over previous
import jax
import jax.numpy as jnp
from jax import lax
from jax.experimental import pallas as pl
from jax.experimental.pallas import tpu as pltpu
from jax.experimental.pallas import tpu_sc as plsc

N = 10000
E = 320000
D = 128
D2 = 2 * D
DE = 16
H = 8
C = 16
SCALE = 0.25

NC = 2
NS = 16
NW = NC * NS
EPW = E // NW
B = 80
NCH = EPW // B
N_PAD = 10240
RPT = N_PAD // NS


def _proj_body(x_ref, w_ref, b_ref, o_ref):
    o_ref[...] = jnp.dot(x_ref[...], w_ref[...],
                         preferred_element_type=jnp.float32) + b_ref[...]


def _proj(x, w, b2d):
    bn = 2000
    grid = N // bn
    row = pl.BlockSpec((bn, D), lambda i: (i, 0))
    return pl.pallas_call(
        _proj_body,
        grid=(grid,),
        in_specs=[row, pl.BlockSpec((D, D), lambda i: (0, 0)),
                  pl.BlockSpec((1, D), lambda i: (0, 0))],
        out_specs=row,
        out_shape=jax.ShapeDtypeStruct((N, D), jnp.float32),
    )(x, w, b2d)


def _qkvs_body(h_ref, wq_ref, wk_ref, wv_ref, ws_ref, bq_ref, bk_ref, bv_ref,
               bs_ref, q_ref, kv_ref, s_ref):
    hb = h_ref[...]
    q_ref[...] = jnp.dot(hb, wq_ref[...], preferred_element_type=jnp.float32) + bq_ref[...]
    k = jnp.dot(hb, wk_ref[...], preferred_element_type=jnp.float32) + bk_ref[...]
    v = jnp.dot(hb, wv_ref[...], preferred_element_type=jnp.float32) + bv_ref[...]
    kv_ref[...] = jnp.concatenate([k, v], axis=1)
    s_ref[...] = jnp.dot(hb, ws_ref[...], preferred_element_type=jnp.float32) + bs_ref[...]


def _qkvs(h, wq, wk, wv, ws, bq, bk, bv, bs):
    bn = 2000
    grid = N // bn
    row = pl.BlockSpec((bn, D), lambda i: (i, 0))
    mat = pl.BlockSpec((D, D), lambda i: (0, 0))
    bias = pl.BlockSpec((1, D), lambda i: (0, 0))
    return pl.pallas_call(
        _qkvs_body,
        grid=(grid,),
        in_specs=[row, mat, mat, mat, mat, bias, bias, bias, bias],
        out_specs=[row, pl.BlockSpec((bn, D2), lambda i: (i, 0)), row],
        out_shape=[jax.ShapeDtypeStruct((N, D), jnp.float32),
                   jax.ShapeDtypeStruct((N, D2), jnp.float32),
                   jax.ShapeDtypeStruct((N, D), jnp.float32)],
    )(h, wq, wk, wv, ws, bq, bk, bv, bs)


def _edge_body(qi_ref, kvj_ref, ea_ref, ew_ref, we_ref, eb_ref, g_ref, s_ref,
               m_ref, w_ref):
    w2 = jnp.dot(ew_ref[...], we_ref[...], preferred_element_type=jnp.float32)
    b2 = jnp.dot(eb_ref[...], we_ref[...], preferred_element_type=jnp.float32)
    eeb = jnp.dot(ea_ref[...], w2, preferred_element_type=jnp.float32) + b2
    kvj = kvj_ref[...]
    kj = kvj[:, :D] + eeb
    vj = kvj[:, D:] + eeb
    t = qi_ref[...] * kj
    alpha = jnp.dot(t, g_ref[...], preferred_element_type=jnp.float32)
    ex = jnp.exp(alpha * SCALE)
    exb = jnp.dot(ex, s_ref[...], preferred_element_type=jnp.float32)
    w_ref[...] = exb
    m_ref[...] = exb * vj


def _edge_compute(qi, kvj, edge_attr, edge_W, We_l, edge_b2, gsel, ssel):
    be = 2000
    grid = E // be
    row = pl.BlockSpec((be, D), lambda i: (i, 0))
    return pl.pallas_call(
        _edge_body,
        grid=(grid,),
        in_specs=[row,
                  pl.BlockSpec((be, D2), lambda i: (i, 0)),
                  pl.BlockSpec((be, DE), lambda i: (i, 0)),
                  pl.BlockSpec((DE, D), lambda i: (0, 0)),
                  pl.BlockSpec((D, D), lambda i: (0, 0)),
                  pl.BlockSpec((1, D), lambda i: (0, 0)),
                  pl.BlockSpec((D, H), lambda i: (0, 0)),
                  pl.BlockSpec((H, D), lambda i: (0, 0))],
        out_specs=[row, row],
        out_shape=[jax.ShapeDtypeStruct((E, D), jnp.float32),
                   jax.ShapeDtypeStruct((E, D), jnp.float32)],
    )(qi, kvj, edge_attr, edge_W, We_l, edge_b2, gsel, ssel)


def _norm_body(a0_ref, a1_ref, d0_ref, d1_ref, skip_ref, h_ref, o_ref):
    denb = d0_ref[...] + d1_ref[...]
    agg = (a0_ref[...] + a1_ref[...]) / (denb + 1e-16) + skip_ref[...]
    o_ref[...] = h_ref[...] + jnp.maximum(agg, 0.0)


def _norm_update(acc0, acc1, den0, den1, skip, h):
    bn = 2000
    grid = N // bn
    row = pl.BlockSpec((bn, D), lambda i: (i, 0))
    return pl.pallas_call(
        _norm_body,
        grid=(grid,),
        in_specs=[row, row, row, row, row, row],
        out_specs=row,
        out_shape=jax.ShapeDtypeStruct((N, D), jnp.float32),
    )(acc0, acc1, den0, den1, skip, h)


def _final_body(h_ref, g_ref, b_ref, ow_ref, ob_ref, o_ref, acc_ref):
    i = pl.program_id(0)
    nblk = pl.num_programs(0)
    hb = h_ref[...]
    mu = jnp.mean(hb, axis=1, keepdims=True)
    var = jnp.mean((hb - mu) ** 2, axis=1, keepdims=True)
    hn = (hb - mu) / jnp.sqrt(var + 1e-5) * g_ref[...] + b_ref[...]
    psum = jnp.sum(hn, axis=0, keepdims=True)

    @pl.when(i == 0)
    def _():
        acc_ref[...] = jnp.zeros_like(acc_ref)

    acc_ref[...] += psum

    @pl.when(i == nblk - 1)
    def _():
        o_ref[...] = jnp.dot(acc_ref[...] * (1.0 / N), ow_ref[...],
                             preferred_element_type=jnp.float32) + ob_ref[...]


def _final(h, ln_g2d, ln_b2d, out_W, out_b2d):
    bn = 2000
    grid = N // bn
    return pl.pallas_call(
        _final_body,
        grid=(grid,),
        in_specs=[pl.BlockSpec((bn, D), lambda i: (i, 0)),
                  pl.BlockSpec((1, D), lambda i: (0, 0)),
                  pl.BlockSpec((1, D), lambda i: (0, 0)),
                  pl.BlockSpec((D, D), lambda i: (0, 0)),
                  pl.BlockSpec((1, D), lambda i: (0, 0))],
        out_specs=pl.BlockSpec((1, D), lambda i: (0, 0)),
        out_shape=jax.ShapeDtypeStruct((1, D), jnp.float32),
        scratch_shapes=[pltpu.VMEM((1, D), jnp.float32)],
    )(h, ln_g2d, ln_b2d, out_W, out_b2d)


def _sc_gather_body(kvT, qT, src, dst, kvj_out, qi_out,
                    idxs, idxd, kvb, qb, sg0, sg1, sw0, sw1):
    cid = lax.axis_index("c")
    sid = lax.axis_index("s")
    wid = cid * NS + sid
    base = wid * EPW
    sg = (sg0, sg1)
    sw = (sw0, sw1)

    def issue_gather(c, p):
        eb = base + c * B
        pltpu.sync_copy(src.at[pl.ds(eb, B)], idxs.at[p])
        pltpu.sync_copy(dst.at[pl.ds(eb, B)], idxd.at[p])
        pltpu.async_copy(kvT.at[idxs.at[p]], kvb.at[p], sg[p])
        pltpu.async_copy(qT.at[idxd.at[p]], qb.at[p], sg[p])

    def wait_gather(p):
        pltpu.make_async_copy(kvT.at[idxs.at[p]], kvb.at[p], sg[p]).wait()
        pltpu.make_async_copy(qT.at[idxd.at[p]], qb.at[p], sg[p]).wait()

    def issue_write(c, p):
        eb = base + c * B
        pltpu.async_copy(kvb.at[p], kvj_out.at[pl.ds(eb, B)], sw[p])
        pltpu.async_copy(qb.at[p], qi_out.at[pl.ds(eb, B)], sw[p])

    def wait_write(p):
        pltpu.make_async_copy(kvb.at[p], kvj_out.at[pl.ds(base, B)], sw[p]).wait()
        pltpu.make_async_copy(qb.at[p], qi_out.at[pl.ds(base, B)], sw[p]).wait()

    issue_gather(0, 0)

    @pl.loop(0, NCH - 1, step=2)
    def _(c):
        @pl.when(c > 0)
        def _():
            wait_write(1)

        issue_gather(c + 1, 1)
        wait_gather(0)
        issue_write(c, 0)
        wait_gather(1)
        issue_write(c + 1, 1)
        wait_write(0)
        issue_gather(c + 2, 0)

    wait_gather(0)
    wait_write(1)
    issue_write(NCH - 1, 0)
    wait_write(0)


def _sc_gather(kvT, qT, src, dst):
    mesh = plsc.VectorSubcoreMesh(core_axis_name="c", subcore_axis_name="s")
    f = pl.kernel(
        _sc_gather_body,
        out_type=[jax.ShapeDtypeStruct((E, D2), jnp.float32),
                  jax.ShapeDtypeStruct((E, D), jnp.float32)],
        mesh=mesh,
        scratch_types=[
            pltpu.VMEM((2, B), jnp.int32),
            pltpu.VMEM((2, B), jnp.int32),
            pltpu.VMEM((2, B, D2), jnp.float32),
            pltpu.VMEM((2, B, D), jnp.float32),
            pltpu.SemaphoreType.DMA,
            pltpu.SemaphoreType.DMA,
            pltpu.SemaphoreType.DMA,
            pltpu.SemaphoreType.DMA,
        ],
    )
    return f(kvT, qT, src, dst)


def _sc_scatter_body(msg, w8, dst, znode, acc_out, den_out,
                     tab_sh, idx, mb, sl):
    cid = lax.axis_index("c")
    sid = lax.axis_index("s")
    wid = cid * NS + sid
    base = wid * EPW
    r0 = sid * RPT
    off = cid * N_PAD + r0

    def accumulate(src_arr, out_arr):
        pltpu.sync_copy(znode.at[pl.ds(r0, RPT)], tab_sh.at[pl.ds(r0, RPT)])
        plsc.subcore_barrier()

        def chunk(ch, carry):
            eb = base + ch * B
            d1 = pltpu.async_copy(dst.at[pl.ds(eb, B)], idx.at[0], sl)
            d2 = pltpu.async_copy(src_arr.at[pl.ds(eb, B)], mb, sl)
            d1.wait()
            d2.wait()
            pltpu.sync_copy(mb, tab_sh.at[idx.at[0]], add=True)
            return carry

        lax.fori_loop(0, NCH, chunk, 0)
        plsc.subcore_barrier()
        pltpu.sync_copy(tab_sh.at[pl.ds(r0, RPT)], out_arr.at[pl.ds(off, RPT)])
        plsc.subcore_barrier()

    accumulate(msg, acc_out)
    accumulate(w8, den_out)


def _sc_scatter(msg, w8, dst, znode):
    mesh = plsc.VectorSubcoreMesh(core_axis_name="c", subcore_axis_name="s")
    f = pl.kernel(
        _sc_scatter_body,
        out_type=[jax.ShapeDtypeStruct((NC * N_PAD, D), jnp.float32),
                  jax.ShapeDtypeStruct((NC * N_PAD, D), jnp.float32)],
        mesh=mesh,
        scratch_types=[
            pltpu.VMEM_SHARED((N_PAD, D), jnp.float32),
            pltpu.VMEM((1, B), jnp.int32),
            pltpu.VMEM((B, D), jnp.float32),
            pltpu.SemaphoreType.DMA,
        ],
    )
    accP, denP = f(msg, w8, dst, znode)
    return accP.reshape(NC, N_PAD, D), denP.reshape(NC, N_PAD, D)


def kernel(x, edge_index, edge_attr, node_W, node_b, edge_W, edge_b, Wq, bq,
           Wk, bk, Wv, bv, We, Wskip, bskip, ln_g, ln_b, out_W, out_b):
    src = edge_index[0]
    dst = edge_index[1]

    lane = jnp.arange(D, dtype=jnp.int32)
    head = jnp.arange(H, dtype=jnp.int32)
    gsel = (lane[:, None] // C == head[None, :]).astype(jnp.float32)
    ssel = gsel.T.copy()

    znode = jnp.zeros((N_PAD, D), jnp.float32)

    node_b2 = node_b.reshape(1, D)
    edge_b2 = edge_b.reshape(1, D)

    h = _proj(x, node_W, node_b2)

    for l in range(2):
        q, kv, skip = _qkvs(h, Wq[l], Wk[l], Wv[l], Wskip[l],
                            bq[l].reshape(1, D), bk[l].reshape(1, D),
                            bv[l].reshape(1, D), bskip[l].reshape(1, D))
        kvj, qi = _sc_gather(kv, q, src, dst)
        msg, w8 = _edge_compute(qi, kvj, edge_attr, edge_W, We[l], edge_b2,
                                gsel, ssel)
        accP, denP = _sc_scatter(msg, w8, dst, znode)
        h = _norm_update(accP[0, :N], accP[1, :N], denP[0, :N], denP[1, :N],
                         skip, h)

    return _final(h, ln_g.reshape(1, D), ln_b.reshape(1, D), out_W,
                  out_b.reshape(1, D))


# speedup vs baseline: 42.9641x; 1.1708x over previous
import jax
import jax.numpy as jnp
from jax import lax
from jax.experimental import pallas as pl
from jax.experimental.pallas import tpu as pltpu
from jax.experimental.pallas import tpu_sc as plsc

N = 10000
E = 320000
D = 128
D2 = 2 * D
DE = 16
H = 8
C = 16
SCALE = 0.25

NC = 2
NS = 16
NW = NC * NS
EPW = E // NW
B = 80
NCH = EPW // B
N_PAD = 10240
RPT = N_PAD // NS


def _proj_body(x_ref, w_ref, b_ref, o_ref):
    o_ref[...] = jnp.dot(x_ref[...], w_ref[...],
                         preferred_element_type=jnp.float32) + b_ref[...]


def _proj(x, w, b2d):
    bn = 2000
    grid = N // bn
    row = pl.BlockSpec((bn, D), lambda i: (i, 0))
    return pl.pallas_call(
        _proj_body,
        grid=(grid,),
        in_specs=[row, pl.BlockSpec((D, D), lambda i: (0, 0)),
                  pl.BlockSpec((1, D), lambda i: (0, 0))],
        out_specs=row,
        out_shape=jax.ShapeDtypeStruct((N, D), jnp.float32),
    )(x, w, b2d)


def _qkvs_body(h_ref, wq_ref, wk_ref, wv_ref, ws_ref, bq_ref, bk_ref, bv_ref,
               bs_ref, q_ref, kv_ref, s_ref):
    hb = h_ref[...]
    q_ref[...] = jnp.dot(hb, wq_ref[...], preferred_element_type=jnp.float32) + bq_ref[...]
    k = jnp.dot(hb, wk_ref[...], preferred_element_type=jnp.float32) + bk_ref[...]
    v = jnp.dot(hb, wv_ref[...], preferred_element_type=jnp.float32) + bv_ref[...]
    kv_ref[...] = jnp.concatenate([k, v], axis=1)
    s_ref[...] = jnp.dot(hb, ws_ref[...], preferred_element_type=jnp.float32) + bs_ref[...]


def _qkvs(h, wq, wk, wv, ws, bq, bk, bv, bs):
    bn = 2000
    grid = N // bn
    row = pl.BlockSpec((bn, D), lambda i: (i, 0))
    mat = pl.BlockSpec((D, D), lambda i: (0, 0))
    bias = pl.BlockSpec((1, D), lambda i: (0, 0))
    return pl.pallas_call(
        _qkvs_body,
        grid=(grid,),
        in_specs=[row, mat, mat, mat, mat, bias, bias, bias, bias],
        out_specs=[row, pl.BlockSpec((bn, D2), lambda i: (i, 0)), row],
        out_shape=[jax.ShapeDtypeStruct((N, D), jnp.float32),
                   jax.ShapeDtypeStruct((N, D2), jnp.float32),
                   jax.ShapeDtypeStruct((N, D), jnp.float32)],
    )(h, wq, wk, wv, ws, bq, bk, bv, bs)


def _edge_body(qi_ref, kvj_ref, ea_ref, ew_ref, we_ref, eb_ref, g_ref, s_ref,
               m_ref, w_ref):
    w2 = jnp.dot(ew_ref[...], we_ref[...], preferred_element_type=jnp.float32)
    b2 = jnp.dot(eb_ref[...], we_ref[...], preferred_element_type=jnp.float32)
    eeb = jnp.dot(ea_ref[...], w2, preferred_element_type=jnp.float32) + b2
    kvj = kvj_ref[...]
    kj = kvj[:, :D] + eeb
    vj = kvj[:, D:] + eeb
    t = qi_ref[...] * kj
    alpha = jnp.dot(t, g_ref[...], preferred_element_type=jnp.float32)
    ex = jnp.exp(alpha * SCALE)
    exb = jnp.dot(ex, s_ref[...], preferred_element_type=jnp.float32)
    w_ref[...] = exb
    m_ref[...] = exb * vj


def _edge_compute(qi, kvj, edge_attr, edge_W, We_l, edge_b2, gsel, ssel):
    be = 2000
    grid = E // be
    row = pl.BlockSpec((be, D), lambda i: (i, 0))
    return pl.pallas_call(
        _edge_body,
        grid=(grid,),
        in_specs=[row,
                  pl.BlockSpec((be, D2), lambda i: (i, 0)),
                  pl.BlockSpec((be, DE), lambda i: (i, 0)),
                  pl.BlockSpec((DE, D), lambda i: (0, 0)),
                  pl.BlockSpec((D, D), lambda i: (0, 0)),
                  pl.BlockSpec((1, D), lambda i: (0, 0)),
                  pl.BlockSpec((D, H), lambda i: (0, 0)),
                  pl.BlockSpec((H, D), lambda i: (0, 0))],
        out_specs=[row, row],
        out_shape=[jax.ShapeDtypeStruct((E, D), jnp.float32),
                   jax.ShapeDtypeStruct((E, D), jnp.float32)],
    )(qi, kvj, edge_attr, edge_W, We_l, edge_b2, gsel, ssel)


def _norm_body(a0_ref, a1_ref, d0_ref, d1_ref, skip_ref, h_ref, o_ref):
    denb = d0_ref[...] + d1_ref[...]
    agg = (a0_ref[...] + a1_ref[...]) / (denb + 1e-16) + skip_ref[...]
    o_ref[...] = h_ref[...] + jnp.maximum(agg, 0.0)


def _norm_update(acc0, acc1, den0, den1, skip, h):
    bn = 2000
    grid = N // bn
    row = pl.BlockSpec((bn, D), lambda i: (i, 0))
    return pl.pallas_call(
        _norm_body,
        grid=(grid,),
        in_specs=[row, row, row, row, row, row],
        out_specs=row,
        out_shape=jax.ShapeDtypeStruct((N, D), jnp.float32),
    )(acc0, acc1, den0, den1, skip, h)


def _final_body(h_ref, g_ref, b_ref, ow_ref, ob_ref, o_ref, acc_ref):
    i = pl.program_id(0)
    nblk = pl.num_programs(0)
    hb = h_ref[...]
    mu = jnp.mean(hb, axis=1, keepdims=True)
    var = jnp.mean((hb - mu) ** 2, axis=1, keepdims=True)
    hn = (hb - mu) / jnp.sqrt(var + 1e-5) * g_ref[...] + b_ref[...]
    psum = jnp.sum(hn, axis=0, keepdims=True)

    @pl.when(i == 0)
    def _():
        acc_ref[...] = jnp.zeros_like(acc_ref)

    acc_ref[...] += psum

    @pl.when(i == nblk - 1)
    def _():
        o_ref[...] = jnp.dot(acc_ref[...] * (1.0 / N), ow_ref[...],
                             preferred_element_type=jnp.float32) + ob_ref[...]


def _final(h, ln_g2d, ln_b2d, out_W, out_b2d):
    bn = 2000
    grid = N // bn
    return pl.pallas_call(
        _final_body,
        grid=(grid,),
        in_specs=[pl.BlockSpec((bn, D), lambda i: (i, 0)),
                  pl.BlockSpec((1, D), lambda i: (0, 0)),
                  pl.BlockSpec((1, D), lambda i: (0, 0)),
                  pl.BlockSpec((D, D), lambda i: (0, 0)),
                  pl.BlockSpec((1, D), lambda i: (0, 0))],
        out_specs=pl.BlockSpec((1, D), lambda i: (0, 0)),
        out_shape=jax.ShapeDtypeStruct((1, D), jnp.float32),
        scratch_shapes=[pltpu.VMEM((1, D), jnp.float32)],
    )(h, ln_g2d, ln_b2d, out_W, out_b2d)


def _sc_gather_body(kvT, qT, src, dst, kvj_out, qi_out,
                    idxs, idxd, kvb, qb, sg0, sg1, sw0, sw1):
    cid = lax.axis_index("c")
    sid = lax.axis_index("s")
    wid = cid * NS + sid
    base = wid * EPW
    sg = (sg0, sg1)
    sw = (sw0, sw1)

    def issue_gather(c, p):
        eb = base + c * B
        pltpu.sync_copy(src.at[pl.ds(eb, B)], idxs.at[p])
        pltpu.sync_copy(dst.at[pl.ds(eb, B)], idxd.at[p])
        pltpu.async_copy(kvT.at[idxs.at[p]], kvb.at[p], sg[p])
        pltpu.async_copy(qT.at[idxd.at[p]], qb.at[p], sg[p])

    def wait_gather(p):
        pltpu.make_async_copy(kvT.at[idxs.at[p]], kvb.at[p], sg[p]).wait()
        pltpu.make_async_copy(qT.at[idxd.at[p]], qb.at[p], sg[p]).wait()

    def issue_write(c, p):
        eb = base + c * B
        pltpu.async_copy(kvb.at[p], kvj_out.at[pl.ds(eb, B)], sw[p])
        pltpu.async_copy(qb.at[p], qi_out.at[pl.ds(eb, B)], sw[p])

    def wait_write(p):
        pltpu.make_async_copy(kvb.at[p], kvj_out.at[pl.ds(base, B)], sw[p]).wait()
        pltpu.make_async_copy(qb.at[p], qi_out.at[pl.ds(base, B)], sw[p]).wait()

    issue_gather(0, 0)

    @pl.loop(0, NCH - 1, step=2)
    def _(c):
        @pl.when(c > 0)
        def _():
            wait_write(1)

        issue_gather(c + 1, 1)
        wait_gather(0)
        issue_write(c, 0)
        wait_gather(1)
        issue_write(c + 1, 1)
        wait_write(0)
        issue_gather(c + 2, 0)

    wait_gather(0)
    wait_write(1)
    issue_write(NCH - 1, 0)
    wait_write(0)


def _sc_gather(kvT, qT, src, dst):
    mesh = plsc.VectorSubcoreMesh(core_axis_name="c", subcore_axis_name="s")
    f = pl.kernel(
        _sc_gather_body,
        out_type=[jax.ShapeDtypeStruct((E, D2), jnp.float32),
                  jax.ShapeDtypeStruct((E, D), jnp.float32)],
        mesh=mesh,
        scratch_types=[
            pltpu.VMEM((2, B), jnp.int32),
            pltpu.VMEM((2, B), jnp.int32),
            pltpu.VMEM((2, B, D2), jnp.float32),
            pltpu.VMEM((2, B, D), jnp.float32),
            pltpu.SemaphoreType.DMA,
            pltpu.SemaphoreType.DMA,
            pltpu.SemaphoreType.DMA,
            pltpu.SemaphoreType.DMA,
        ],
    )
    return f(kvT, qT, src, dst)


def _sc_scatter_body(msg, w8, dst, znode, acc_out, den_out,
                     tab_sh, idx, mb, sl0, sl1):
    cid = lax.axis_index("c")
    sid = lax.axis_index("s")
    wid = cid * NS + sid
    base = wid * EPW
    r0 = sid * RPT
    off = cid * N_PAD + r0
    sl = (sl0, sl1)

    def accumulate(src_arr, out_arr):
        pltpu.sync_copy(znode.at[pl.ds(r0, RPT)], tab_sh.at[pl.ds(r0, RPT)])
        plsc.subcore_barrier()

        def issue_load(c, p):
            eb = base + c * B
            pltpu.async_copy(dst.at[pl.ds(eb, B)], idx.at[p], sl[p])
            pltpu.async_copy(src_arr.at[pl.ds(eb, B)], mb.at[p], sl[p])

        def wait_load(p):
            pltpu.make_async_copy(dst.at[pl.ds(base, B)], idx.at[p], sl[p]).wait()
            pltpu.make_async_copy(src_arr.at[pl.ds(base, B)], mb.at[p], sl[p]).wait()

        def scatter(p):
            pltpu.sync_copy(mb.at[p], tab_sh.at[idx.at[p]], add=True)

        issue_load(0, 0)

        @pl.loop(0, NCH - 1, step=2)
        def _(c):
            issue_load(c + 1, 1)
            wait_load(0)
            scatter(0)
            issue_load(c + 2, 0)
            wait_load(1)
            scatter(1)

        wait_load(0)
        scatter(0)
        plsc.subcore_barrier()
        pltpu.sync_copy(tab_sh.at[pl.ds(r0, RPT)], out_arr.at[pl.ds(off, RPT)])
        plsc.subcore_barrier()

    accumulate(msg, acc_out)
    accumulate(w8, den_out)


def _sc_scatter(msg, w8, dst, znode):
    mesh = plsc.VectorSubcoreMesh(core_axis_name="c", subcore_axis_name="s")
    f = pl.kernel(
        _sc_scatter_body,
        out_type=[jax.ShapeDtypeStruct((NC * N_PAD, D), jnp.float32),
                  jax.ShapeDtypeStruct((NC * N_PAD, D), jnp.float32)],
        mesh=mesh,
        scratch_types=[
            pltpu.VMEM_SHARED((N_PAD, D), jnp.float32),
            pltpu.VMEM((2, B), jnp.int32),
            pltpu.VMEM((2, B, D), jnp.float32),
            pltpu.SemaphoreType.DMA,
            pltpu.SemaphoreType.DMA,
        ],
    )
    accP, denP = f(msg, w8, dst, znode)
    return accP.reshape(NC, N_PAD, D), denP.reshape(NC, N_PAD, D)


def kernel(x, edge_index, edge_attr, node_W, node_b, edge_W, edge_b, Wq, bq,
           Wk, bk, Wv, bv, We, Wskip, bskip, ln_g, ln_b, out_W, out_b):
    src = edge_index[0]
    dst = edge_index[1]

    lane = jnp.arange(D, dtype=jnp.int32)
    head = jnp.arange(H, dtype=jnp.int32)
    gsel = (lane[:, None] // C == head[None, :]).astype(jnp.float32)
    ssel = gsel.T.copy()

    znode = jnp.zeros((N_PAD, D), jnp.float32)

    node_b2 = node_b.reshape(1, D)
    edge_b2 = edge_b.reshape(1, D)

    h = _proj(x, node_W, node_b2)

    for l in range(2):
        q, kv, skip = _qkvs(h, Wq[l], Wk[l], Wv[l], Wskip[l],
                            bq[l].reshape(1, D), bk[l].reshape(1, D),
                            bv[l].reshape(1, D), bskip[l].reshape(1, D))
        kvj, qi = _sc_gather(kv, q, src, dst)
        msg, w8 = _edge_compute(qi, kvj, edge_attr, edge_W, We[l], edge_b2,
                                gsel, ssel)
        accP, denP = _sc_scatter(msg, w8, dst, znode)
        h = _norm_update(accP[0, :N], accP[1, :N], denP[0, :N], denP[1, :N],
                         skip, h)

    return _final(h, ln_g.reshape(1, D), ln_b.reshape(1, D), out_W,
                  out_b.reshape(1, D))
